# Initial kernel scaffold; baseline (speedup 1.0000x reference)
import jax
import jax.numpy as jnp
from jax.experimental import pallas as pl


def kernel(conv1_1_w, conv1_1_b, conv1_2_w, conv1_2_b, conv2_1_w, conv2_1_b, conv2_2_w, conv2_2_b, conv3_1_w, conv3_1_b, conv3_2_w, conv3_2_b, conv3_3_w, conv3_3_b, conv4_1_w, conv4_1_b, conv4_2_w, conv4_2_b, conv4_3_w, conv4_3_b, conv5_1_w, conv5_1_b, conv5_2_w, conv5_2_b, conv5_3_w, conv5_3_b, fc6_w, fc6_b, fc7_w, fc7_b, fc_final_w, fc_final_b, ct_conv1_1_w, ct_conv1_1_b, ct_conv1_2_w, ct_conv1_2_b, ct_conv2_1_w, ct_conv2_1_b, ct_conv3_1_w, ct_conv3_1_b, ct_conv4_1_w, ct_conv4_1_b, ct_conv5_1_w, ct_conv5_1_b, ct_fc1_w, ct_fc1_b, ct_final_w, ct_final_b, x):
    raise NotImplementedError("write your pallas kernel here")


# speedup vs baseline: 3.7031x; 3.7031x over previous
import functools

import jax
import jax.numpy as jnp
from jax.experimental import pallas as pl
from jax.experimental.pallas import tpu as pltpu

_F32 = jnp.float32
_BF16 = jnp.bfloat16


def _conv3_body(x_ref, w_ref, b_ref, o_ref, *, tr, nchunks, relu):
    _, H, W, Cin = x_ref.shape
    _, Ho, Wo, Cout = o_ref.shape

    def do_chunk(s):
        a = jnp.concatenate(
            [x_ref[0, pl.ds(s + kh, tr)] for kh in range(3)], axis=-1)
        p = jnp.dot(a.reshape(tr * W, 3 * Cin), w_ref[...],
                    preferred_element_type=_F32)
        p = p.reshape(tr, W, 3 * Cout)
        v = (p[:, 0:Wo, 0:Cout]
             + p[:, 1:1 + Wo, Cout:2 * Cout]
             + p[:, 2:2 + Wo, 2 * Cout:3 * Cout])
        v = v + b_ref[...]
        if relu:
            v = jnp.maximum(v, 0.0)
        o_ref[0, pl.ds(s, tr)] = v.astype(o_ref.dtype)

    if nchunks == 1:
        do_chunk(0)
    else:
        def body(i, carry):
            do_chunk(jnp.minimum(i * tr, Ho - tr))
            return carry
        jax.lax.fori_loop(0, nchunks, body, 0)


def _pick_tr(Ho, W, cout3):
    cap = max(1, 3_300_000 // (W * cout3 * 4))
    tr = min(Ho, cap, 256)
    if tr >= 8 and tr < Ho:
        tr = (tr // 8) * 8
    return tr, pl.cdiv(Ho, tr)


def _conv3x3(x, w, b, relu=True):
    N, H, W, Cin = x.shape
    Cout = w.shape[-1]
    Ho, Wo = H - 2, W - 2
    wk = w.astype(_BF16).transpose(0, 2, 1, 3).reshape(3 * Cin, 3 * Cout)
    tr, nchunks = _pick_tr(Ho, W, 3 * Cout)
    return pl.pallas_call(
        functools.partial(_conv3_body, tr=tr, nchunks=nchunks, relu=relu),
        out_shape=jax.ShapeDtypeStruct((N, Ho, Wo, Cout), _BF16),
        grid=(N,),
        in_specs=[
            pl.BlockSpec((1, H, W, Cin), lambda n: (n, 0, 0, 0)),
            pl.BlockSpec((3 * Cin, 3 * Cout), lambda n: (0, 0)),
            pl.BlockSpec((1, Cout), lambda n: (0, 0)),
        ],
        out_specs=pl.BlockSpec((1, Ho, Wo, Cout), lambda n: (n, 0, 0, 0)),
        compiler_params=pltpu.CompilerParams(
            dimension_semantics=("parallel",),
            vmem_limit_bytes=64 * 1024 * 1024),
    )(x.astype(_BF16), wk, b.reshape(1, Cout).astype(_F32))


def _mm_body(a_ref, b_ref, bias_ref, o_ref, acc_ref, *, relu, gk):
    part = jnp.dot(a_ref[...], b_ref[...], preferred_element_type=_F32)
    if gk == 1:
        v = part + bias_ref[...]
        if relu:
            v = jnp.maximum(v, 0.0)
        o_ref[...] = v.astype(o_ref.dtype)
        return

    @pl.when(pl.program_id(2) == 0)
    def _():
        acc_ref[...] = jnp.zeros_like(acc_ref)

    acc_ref[...] += part

    @pl.when(pl.program_id(2) == gk - 1)
    def _():
        v = acc_ref[...] + bias_ref[...]
        if relu:
            v = jnp.maximum(v, 0.0)
        o_ref[...] = v.astype(o_ref.dtype)


def _matmul(a, b, bias, relu, out_dtype=_BF16):
    M, K = a.shape
    N = b.shape[1]
    tm = M if M <= 1024 else 1024
    tn = min(N, 512)
    tk = K
    for s in range(1, 65):
        if K % s == 0 and (K // s) * tn * 2 <= 4 * 1024 * 1024:
            tk = K // s
            break
    gm, gn, gk = pl.cdiv(M, tm), pl.cdiv(N, tn), K // tk
    scratch = [pltpu.VMEM((tm, tn), _F32)] if gk > 1 else [
        pltpu.VMEM((8, 128), _F32)]
    return pl.pallas_call(
        functools.partial(_mm_body, relu=relu, gk=gk),
        out_shape=jax.ShapeDtypeStruct((M, N), out_dtype),
        grid=(gm, gn, gk),
        in_specs=[
            pl.BlockSpec((tm, tk), lambda i, j, k: (i, k)),
            pl.BlockSpec((tk, tn), lambda i, j, k: (k, j)),
            pl.BlockSpec((1, tn), lambda i, j, k: (0, j)),
        ],
        out_specs=pl.BlockSpec((tm, tn), lambda i, j, k: (i, j)),
        scratch_shapes=scratch,
        compiler_params=pltpu.CompilerParams(
            dimension_semantics=("parallel", "parallel", "arbitrary"),
            vmem_limit_bytes=48 * 1024 * 1024),
    )(a.astype(_BF16), b.astype(_BF16), bias.reshape(1, N).astype(_F32))


def _pool_body(x_ref, o_ref):
    c = o_ref.shape[-1]
    m = jnp.maximum(x_ref[:, 0], x_ref[:, 1])
    o_ref[...] = jnp.maximum(m[..., :c], m[..., c:])


def _maxpool(x):
    N, H, W, C = x.shape
    Hp, Wp = (H + 1) // 2, (W + 1) // 2
    xp = jnp.pad(x, ((0, 0), (0, 2 * Hp - H), (0, 2 * Wp - W), (0, 0)),
                 constant_values=-jnp.inf)
    xr = xp.reshape(N * Hp, 2, Wp, 2 * C)
    out = pl.pallas_call(
        _pool_body,
        out_shape=jax.ShapeDtypeStruct((N * Hp, Wp, C), x.dtype),
        grid=(N,),
        in_specs=[pl.BlockSpec((Hp, 2, Wp, 2 * C), lambda n: (n, 0, 0, 0))],
        out_specs=pl.BlockSpec((Hp, Wp, C), lambda n: (n, 0, 0)),
        compiler_params=pltpu.CompilerParams(
            dimension_semantics=("parallel",)),
    )(xr)
    return out.reshape(N, Hp, Wp, C)


def _softmax_body(x_ref, o_ref):
    x = x_ref[...].astype(_F32)
    m = jnp.max(x, axis=-1, keepdims=True)
    e = jnp.exp(x - m)
    o_ref[...] = e / jnp.sum(e, axis=-1, keepdims=True)


def _softmax_lanes(x):
    return pl.pallas_call(
        _softmax_body,
        out_shape=jax.ShapeDtypeStruct(x.shape, _F32),
    )(x)


def kernel(conv1_1_w, conv1_1_b, conv1_2_w, conv1_2_b, conv2_1_w, conv2_1_b,
           conv2_2_w, conv2_2_b, conv3_1_w, conv3_1_b, conv3_2_w, conv3_2_b,
           conv3_3_w, conv3_3_b, conv4_1_w, conv4_1_b, conv4_2_w, conv4_2_b,
           conv4_3_w, conv4_3_b, conv5_1_w, conv5_1_b, conv5_2_w, conv5_2_b,
           conv5_3_w, conv5_3_b, fc6_w, fc6_b, fc7_w, fc7_b,
           fc_final_w, fc_final_b, ct_conv1_1_w, ct_conv1_1_b,
           ct_conv1_2_w, ct_conv1_2_b, ct_conv2_1_w, ct_conv2_1_b,
           ct_conv3_1_w, ct_conv3_1_b, ct_conv4_1_w, ct_conv4_1_b,
           ct_conv5_1_w, ct_conv5_1_b, ct_fc1_w, ct_fc1_b,
           ct_final_w, ct_final_b, x):
    N = x.shape[0]
    h = jnp.transpose(x, (0, 2, 3, 1)).astype(_BF16)

    h = _conv3x3(h, conv1_1_w, conv1_1_b)
    h = _conv3x3(h, conv1_2_w, conv1_2_b)
    h = _maxpool(h)
    h = _conv3x3(h, conv2_1_w, conv2_1_b)
    h = _conv3x3(h, conv2_2_w, conv2_2_b)
    h = _maxpool(h)
    h = _conv3x3(h, conv3_1_w, conv3_1_b)
    h = _conv3x3(h, conv3_2_w, conv3_2_b)
    h = _conv3x3(h, conv3_3_w, conv3_3_b)
    h = _maxpool(h)
    h = _conv3x3(h, conv4_1_w, conv4_1_b)
    h = _conv3x3(h, conv4_2_w, conv4_2_b)
    h = _conv3x3(h, conv4_3_w, conv4_3_b)
    h = _conv3x3(h, conv5_1_w, conv5_1_b)
    h = _conv3x3(h, conv5_2_w, conv5_2_b)
    h = _conv3x3(h, conv5_3_w, conv5_3_b)

    flat = h.reshape(N, 7 * 7 * 512)
    h = _matmul(flat, fc6_w.reshape(7 * 7 * 512, 4096), fc6_b, relu=True)
    h = _matmul(h, fc7_w.reshape(4096, 4096), fc7_b, relu=True)
    h = _matmul(h, fc_final_w.reshape(4096, 21), fc_final_b, relu=False)

    patch = jnp.zeros((N, 31, 31, 21), _BF16)
    patch = jax.lax.dynamic_update_slice(
        patch, h.reshape(N, 1, 1, 21), (0, 15, 15, 0))

    patch = _conv3x3(patch, ct_conv1_1_w, ct_conv1_1_b)
    patch = _conv3x3(patch, ct_conv1_2_w, ct_conv1_2_b)
    patch = _conv3x3(patch, ct_conv2_1_w, ct_conv2_1_b)
    patch = _conv3x3(patch, ct_conv3_1_w, ct_conv3_1_b)
    patch = _conv3x3(patch, ct_conv4_1_w, ct_conv4_1_b)
    patch = _conv3x3(patch, ct_conv5_1_w, ct_conv5_1_b)
    patch = _conv3x3(patch, ct_fc1_w, ct_fc1_b)

    logits = _matmul(patch.reshape(N * 17 * 17, 672),
                     ct_final_w.reshape(672, 21), ct_final_b, relu=False)
    soft = _softmax_lanes(logits).reshape(N, 17, 17, 21)
    soft = jnp.transpose(soft, (0, 3, 1, 2))

    bg = soft[:, :, 0:1, 0:1]
    full = jnp.broadcast_to(bg, (N, 21, 53, 53))
    return jax.lax.dynamic_update_slice(
        full, soft[:, :, 1:16, 1:16], (0, 0, 19, 19))


# speedup vs baseline: 5.6525x; 1.5264x over previous
import functools

import jax
import jax.numpy as jnp
from jax.experimental import pallas as pl
from jax.experimental.pallas import tpu as pltpu

_F32 = jnp.float32
_BF16 = jnp.bfloat16


def _conv3_body(x_ref, w_ref, b_ref, o_ref, *, tr, nchunks, relu):
    _, H, Wc, Cin = x_ref.shape
    _, Ho, Ww, Cout = o_ref.shape

    def do_chunk(s):
        rows = [x_ref[0, pl.ds(s + kh, tr)] for kh in range(3)]
        a = jnp.concatenate(
            [rows[kh][:, kw:kw + Ww, :] for kh in range(3) for kw in range(3)],
            axis=-1)
        p = jnp.dot(a.reshape(tr * Ww, 9 * Cin), w_ref[...],
                    preferred_element_type=_F32)
        v = p.reshape(tr, Ww, Cout) + b_ref[...]
        if relu:
            v = jnp.maximum(v, 0.0)
        o_ref[0, pl.ds(s, tr)] = v.astype(o_ref.dtype)

    if nchunks == 1:
        do_chunk(0)
    else:
        def body(i, carry):
            do_chunk(jnp.minimum(i * tr, Ho - tr))
            return carry
        jax.lax.fori_loop(0, nchunks, body, 0)


def _pick_tr(Ho, Ww, cin9, cout):
    cap = max(1, 3_300_000 // (Ww * (cin9 * 2 + cout * 4)))
    tr = min(Ho, cap, 256)
    if tr >= 8 and tr < Ho:
        tr = (tr // 8) * 8
    return tr, pl.cdiv(Ho, tr)


def _pad8(v):
    return -(-v // 8) * 8


def _conv3x3(x, w, b, relu=True, wlog=None):
    N, H, Wa, Cin = x.shape
    if wlog is None:
        wlog = Wa
    Wo = wlog - 2
    Ww = _pad8(Wo)
    Wc = Ww + 2
    if Wa > Wc:
        x = x[:, :, :Wc, :]
    elif Wa < Wc:
        x = jnp.pad(x, ((0, 0), (0, 0), (0, Wc - Wa), (0, 0)))
    Ho = H - 2
    w9 = w.astype(_BF16).reshape(9 * Cin, w.shape[-1])
    Cout = w.shape[-1]
    tr, nchunks = _pick_tr(Ho, Ww, 9 * Cin, Cout)
    out = pl.pallas_call(
        functools.partial(_conv3_body, tr=tr, nchunks=nchunks, relu=relu),
        out_shape=jax.ShapeDtypeStruct((N, Ho, Ww, Cout), _BF16),
        grid=(N,),
        in_specs=[
            pl.BlockSpec((1, H, Wc, Cin), lambda n: (n, 0, 0, 0)),
            pl.BlockSpec((9 * Cin, Cout), lambda n: (0, 0)),
            pl.BlockSpec((1, Cout), lambda n: (0, 0)),
        ],
        out_specs=pl.BlockSpec((1, Ho, Ww, Cout), lambda n: (n, 0, 0, 0)),
        compiler_params=pltpu.CompilerParams(
            dimension_semantics=("parallel",),
            vmem_limit_bytes=64 * 1024 * 1024),
    )(x.astype(_BF16), w9, b.reshape(1, Cout).astype(_F32))
    return out, Wo


def _mm_body(a_ref, b_ref, bias_ref, o_ref, acc_ref, *, relu, gk):
    part = jnp.dot(a_ref[...], b_ref[...], preferred_element_type=_F32)
    if gk == 1:
        v = part + bias_ref[...]
        if relu:
            v = jnp.maximum(v, 0.0)
        o_ref[...] = v.astype(o_ref.dtype)
        return

    @pl.when(pl.program_id(2) == 0)
    def _():
        acc_ref[...] = jnp.zeros_like(acc_ref)

    acc_ref[...] += part

    @pl.when(pl.program_id(2) == gk - 1)
    def _():
        v = acc_ref[...] + bias_ref[...]
        if relu:
            v = jnp.maximum(v, 0.0)
        o_ref[...] = v.astype(o_ref.dtype)


def _matmul(a, b, bias, relu, out_dtype=_BF16):
    M, K = a.shape
    N = b.shape[1]
    tm = M if M <= 1024 else 1024
    tn = min(N, 512)
    tk = K
    for s in range(1, 65):
        if K % s == 0 and (K // s) * tn * 2 <= 4 * 1024 * 1024:
            tk = K // s
            break
    gm, gn, gk = pl.cdiv(M, tm), pl.cdiv(N, tn), K // tk
    scratch = [pltpu.VMEM((tm, tn), _F32)] if gk > 1 else [
        pltpu.VMEM((8, 128), _F32)]
    return pl.pallas_call(
        functools.partial(_mm_body, relu=relu, gk=gk),
        out_shape=jax.ShapeDtypeStruct((M, N), out_dtype),
        grid=(gm, gn, gk),
        in_specs=[
            pl.BlockSpec((tm, tk), lambda i, j, k: (i, k)),
            pl.BlockSpec((tk, tn), lambda i, j, k: (k, j)),
            pl.BlockSpec((1, tn), lambda i, j, k: (0, j)),
        ],
        out_specs=pl.BlockSpec((tm, tn), lambda i, j, k: (i, j)),
        scratch_shapes=scratch,
        compiler_params=pltpu.CompilerParams(
            dimension_semantics=("parallel", "parallel", "arbitrary"),
            vmem_limit_bytes=48 * 1024 * 1024),
    )(a.astype(_BF16), b.astype(_BF16), bias.reshape(1, N).astype(_F32))


def _pool_body(x_ref, o_ref):
    c = o_ref.shape[-1]
    m = jnp.maximum(x_ref[:, 0], x_ref[:, 1])
    o_ref[...] = jnp.maximum(m[..., :c], m[..., c:])


def _maxpool(x):
    N, H, W, C = x.shape
    Hp, Wp = (H + 1) // 2, (W + 1) // 2
    xp = jnp.pad(x, ((0, 0), (0, 2 * Hp - H), (0, 2 * Wp - W), (0, 0)),
                 constant_values=-jnp.inf)
    xr = xp.reshape(N * Hp, 2, Wp, 2 * C)
    out = pl.pallas_call(
        _pool_body,
        out_shape=jax.ShapeDtypeStruct((N * Hp, Wp, C), x.dtype),
        grid=(N,),
        in_specs=[pl.BlockSpec((Hp, 2, Wp, 2 * C), lambda n: (n, 0, 0, 0))],
        out_specs=pl.BlockSpec((Hp, Wp, C), lambda n: (n, 0, 0)),
        compiler_params=pltpu.CompilerParams(
            dimension_semantics=("parallel",)),
    )(xr)
    return out.reshape(N, Hp, Wp, C)


def _softmax_body(x_ref, o_ref):
    x = x_ref[...].astype(_F32)
    m = jnp.max(x, axis=-1, keepdims=True)
    e = jnp.exp(x - m)
    o_ref[...] = e / jnp.sum(e, axis=-1, keepdims=True)


def _softmax_lanes(x):
    return pl.pallas_call(
        _softmax_body,
        out_shape=jax.ShapeDtypeStruct(x.shape, _F32),
    )(x)


def kernel(conv1_1_w, conv1_1_b, conv1_2_w, conv1_2_b, conv2_1_w, conv2_1_b,
           conv2_2_w, conv2_2_b, conv3_1_w, conv3_1_b, conv3_2_w, conv3_2_b,
           conv3_3_w, conv3_3_b, conv4_1_w, conv4_1_b, conv4_2_w, conv4_2_b,
           conv4_3_w, conv4_3_b, conv5_1_w, conv5_1_b, conv5_2_w, conv5_2_b,
           conv5_3_w, conv5_3_b, fc6_w, fc6_b, fc7_w, fc7_b,
           fc_final_w, fc_final_b, ct_conv1_1_w, ct_conv1_1_b,
           ct_conv1_2_w, ct_conv1_2_b, ct_conv2_1_w, ct_conv2_1_b,
           ct_conv3_1_w, ct_conv3_1_b, ct_conv4_1_w, ct_conv4_1_b,
           ct_conv5_1_w, ct_conv5_1_b, ct_fc1_w, ct_fc1_b,
           ct_final_w, ct_final_b, x):
    N = x.shape[0]
    h = jnp.transpose(x, (0, 2, 3, 1)).astype(_BF16)

    h, wl = _conv3x3(h, conv1_1_w, conv1_1_b)
    h, wl = _conv3x3(h, conv1_2_w, conv1_2_b, wlog=wl)
    h = _maxpool(h[:, :, :wl, :])
    h, wl = _conv3x3(h, conv2_1_w, conv2_1_b)
    h, wl = _conv3x3(h, conv2_2_w, conv2_2_b, wlog=wl)
    h = _maxpool(h[:, :, :wl, :])
    h, wl = _conv3x3(h, conv3_1_w, conv3_1_b)
    h, wl = _conv3x3(h, conv3_2_w, conv3_2_b, wlog=wl)
    h, wl = _conv3x3(h, conv3_3_w, conv3_3_b, wlog=wl)
    h = _maxpool(h[:, :, :wl, :])
    h, wl = _conv3x3(h, conv4_1_w, conv4_1_b)
    h, wl = _conv3x3(h, conv4_2_w, conv4_2_b, wlog=wl)
    h, wl = _conv3x3(h, conv4_3_w, conv4_3_b, wlog=wl)
    h, wl = _conv3x3(h, conv5_1_w, conv5_1_b, wlog=wl)
    h, wl = _conv3x3(h, conv5_2_w, conv5_2_b, wlog=wl)
    h, wl = _conv3x3(h, conv5_3_w, conv5_3_b, wlog=wl)

    flat = h[:, :, :7, :].reshape(N, 7 * 7 * 512)
    h = _matmul(flat, fc6_w.reshape(7 * 7 * 512, 4096), fc6_b, relu=True)
    h = _matmul(h, fc7_w.reshape(4096, 4096), fc7_b, relu=True)
    h = _matmul(h, fc_final_w.reshape(4096, 21), fc_final_b, relu=False)

    patch = jnp.zeros((N, 31, 31, 21), _BF16)
    patch = jax.lax.dynamic_update_slice(
        patch, h.reshape(N, 1, 1, 21), (0, 15, 15, 0))

    patch, pw = _conv3x3(patch, ct_conv1_1_w, ct_conv1_1_b)
    patch, pw = _conv3x3(patch, ct_conv1_2_w, ct_conv1_2_b, wlog=pw)
    patch, pw = _conv3x3(patch, ct_conv2_1_w, ct_conv2_1_b, wlog=pw)
    patch, pw = _conv3x3(patch, ct_conv3_1_w, ct_conv3_1_b, wlog=pw)
    patch, pw = _conv3x3(patch, ct_conv4_1_w, ct_conv4_1_b, wlog=pw)
    patch, pw = _conv3x3(patch, ct_conv5_1_w, ct_conv5_1_b, wlog=pw)
    patch, pw = _conv3x3(patch, ct_fc1_w, ct_fc1_b, wlog=pw)

    logits = _matmul(patch[:, :, :17, :].reshape(N * 17 * 17, 672),
                     ct_final_w.reshape(672, 21), ct_final_b, relu=False)
    soft = _softmax_lanes(logits).reshape(N, 17, 17, 21)
    soft = jnp.transpose(soft, (0, 3, 1, 2))

    bg = soft[:, :, 0:1, 0:1]
    full = jnp.broadcast_to(bg, (N, 21, 53, 53))
    return jax.lax.dynamic_update_slice(
        full, soft[:, :, 1:16, 1:16], (0, 0, 19, 19))
